# baseline (speedup 1.0000x reference)
import functools

import jax
import jax.numpy as jnp
from jax import lax
from jax.experimental import pallas as pl
from jax.experimental.pallas import tpu as pltpu
from jax.experimental.pallas import tpu_sc as plsc

N_USER = 1000000
N_ITEM = 1000000
EMB = 16
BATCH = 16384

NUM_CORES = 2
NUM_SUBCORES = 16
NUM_WORKERS = NUM_CORES * NUM_SUBCORES
BPW = BATCH // NUM_WORKERS
GCHUNK = 128
NGCHUNK = BPW // GCHUNK
NBLOCK = BPW // EMB


def _bpr_body(users_hbm, pos_hbm, neg_hbm, uemb_hbm, iemb_hbm,
              rui_hbm, ruj_hbm, loss_hbm,
              idx_u, idx_p, idx_n, u_rows, p_rows, n_rows,
              rui_v, ruj_v, loss_v, sem):
    wid = lax.axis_index("s") * NUM_CORES + lax.axis_index("c")
    base = wid * NGCHUNK

    pltpu.sync_copy(users_hbm.at[pl.ds(base, NGCHUNK)], idx_u)
    pltpu.sync_copy(pos_hbm.at[pl.ds(base, NGCHUNK)], idx_p)
    pltpu.sync_copy(neg_hbm.at[pl.ds(base, NGCHUNK)], idx_n)

    copies = []
    for j in range(NGCHUNK):
        dst = pl.ds(j * GCHUNK, GCHUNK)
        copies.append(pltpu.async_copy(uemb_hbm.at[idx_u.at[j]], u_rows.at[dst], sem))
        copies.append(pltpu.async_copy(iemb_hbm.at[idx_p.at[j]], p_rows.at[dst], sem))
        copies.append(pltpu.async_copy(iemb_hbm.at[idx_n.at[j]], n_rows.at[dst], sem))
    for cp in copies:
        cp.wait()

    def block(bi, loss_acc):
        rows = bi * EMB + lax.iota(jnp.int32, EMB)
        acc_ui = jnp.zeros((EMB,), jnp.float32)
        acc_uj = jnp.zeros((EMB,), jnp.float32)
        for col in range(EMB):
            cols = jnp.full((EMB,), col, jnp.int32)
            u = plsc.load_gather(u_rows, [rows, cols])
            p = plsc.load_gather(p_rows, [rows, cols])
            n = plsc.load_gather(n_rows, [rows, cols])
            acc_ui = acc_ui + u * p
            acc_uj = acc_uj + u * n
            loss_acc = loss_acc + (u * u + p * p + n * n)
        rui_v[pl.ds(bi * EMB, EMB)] = acc_ui
        ruj_v[pl.ds(bi * EMB, EMB)] = acc_uj
        return loss_acc

    loss_acc = lax.fori_loop(0, NBLOCK, block, jnp.zeros((EMB,), jnp.float32))
    loss_v[...] = loss_acc

    out_base = wid * BPW
    pltpu.sync_copy(rui_v, rui_hbm.at[pl.ds(out_base, BPW)])
    pltpu.sync_copy(ruj_v, ruj_hbm.at[pl.ds(out_base, BPW)])
    pltpu.sync_copy(loss_v, loss_hbm.at[wid])


@jax.jit
def _bpr_sc(users2d, pos2d, neg2d, user_emb, item_emb):
    mesh = plsc.VectorSubcoreMesh(core_axis_name="c", subcore_axis_name="s")
    kern = functools.partial(
        pl.kernel,
        mesh=mesh,
        compiler_params=pltpu.CompilerParams(
            needs_layout_passes=False, use_tc_tiling_on_sc=False),
        out_type=[
            jax.ShapeDtypeStruct((BATCH,), jnp.float32),
            jax.ShapeDtypeStruct((BATCH,), jnp.float32),
            jax.ShapeDtypeStruct((NUM_WORKERS, EMB), jnp.float32),
        ],
        scratch_types=[
            pltpu.VMEM((NGCHUNK, GCHUNK), jnp.int32),
            pltpu.VMEM((NGCHUNK, GCHUNK), jnp.int32),
            pltpu.VMEM((NGCHUNK, GCHUNK), jnp.int32),
            pltpu.VMEM((BPW, EMB), jnp.float32),
            pltpu.VMEM((BPW, EMB), jnp.float32),
            pltpu.VMEM((BPW, EMB), jnp.float32),
            pltpu.VMEM((BPW,), jnp.float32),
            pltpu.VMEM((BPW,), jnp.float32),
            pltpu.VMEM((EMB,), jnp.float32),
            pltpu.SemaphoreType.DMA,
        ],
    )(_bpr_body)
    return kern(users2d, pos2d, neg2d, user_emb, item_emb)


def kernel(users, pos_items, neg_items, user_emb, item_emb):
    users2d = users.astype(jnp.int32).reshape(NUM_WORKERS * NGCHUNK, GCHUNK)
    pos2d = pos_items.astype(jnp.int32).reshape(NUM_WORKERS * NGCHUNK, GCHUNK)
    neg2d = neg_items.astype(jnp.int32).reshape(NUM_WORKERS * NGCHUNK, GCHUNK)
    rui, ruj, loss_parts = _bpr_sc(users2d, pos2d, neg2d, user_emb, item_emb)
    return (rui.reshape(BATCH, 1), ruj.reshape(BATCH, 1),
            jnp.sum(loss_parts))
